# baseline (speedup 1.0000x reference)
import jax
import jax.numpy as jnp
from jax.experimental import pallas as pl
from jax.experimental.pallas import tpu as pltpu

BLOCK_N = 2048


def _body(h_ref, w_ref, b_ref, m_ref, o_ref):
    acc = jax.lax.dot_general(
        h_ref[...], w_ref[...],
        dimension_numbers=(((1,), (0,)), ((), ())),
        preferred_element_type=jnp.float32,
    )
    o_ref[...] = acc + b_ref[...] + (1.0 - m_ref[...]) * -1000000000.0


def kernel(hidden, W, b, mask):
    B, H = hidden.shape
    V = W.shape[1]
    b2 = b.reshape(1, V)
    m2 = mask.reshape(1, V)
    grid = (pl.cdiv(V, BLOCK_N),)
    return pl.pallas_call(
        _body,
        grid=grid,
        in_specs=[
            pl.BlockSpec((B, H), lambda j: (0, 0)),
            pl.BlockSpec((H, BLOCK_N), lambda j: (0, j)),
            pl.BlockSpec((1, BLOCK_N), lambda j: (0, j)),
            pl.BlockSpec((1, BLOCK_N), lambda j: (0, j)),
        ],
        out_specs=pl.BlockSpec((B, BLOCK_N), lambda j: (0, j)),
        out_shape=jax.ShapeDtypeStruct((B, V), jnp.float32),
        compiler_params=pltpu.CompilerParams(
            dimension_semantics=("parallel",),
        ),
    )(hidden, W, b2, m2)
